# baseline (speedup 1.0000x reference)
import functools
import jax
import jax.numpy as jnp
from jax import lax
from jax.experimental import pallas as pl
from jax.experimental.pallas import tpu as pltpu
from jax.experimental.pallas import tpu_sc as plsc

BATCH = 16384
HIST = 50
D = 64
VOCAB = 1000000
NC, NS = 2, 16
NW = NC * NS
BBLK = 128
CHUNKS = HIST * (BATCH // BBLK)
PER_W = CHUNKS // NW

ARB = 512
AG = 977
HALF = AG * ARB


def _prep_body(lo_ref, hi_ref, out_ref):
    out_ref[...] = jnp.concatenate([lo_ref[...].T, hi_ref[...].T], axis=1)


@jax.jit
def _prep(lutT):
    return pl.pallas_call(
        _prep_body,
        grid=(AG,),
        in_specs=[
            pl.BlockSpec((D, ARB), lambda i: (0, i)),
            pl.BlockSpec((D, ARB), lambda i: (0, i + AG)),
        ],
        out_specs=pl.BlockSpec((ARB, 128), lambda i: (i, 0)),
        out_shape=jax.ShapeDtypeStruct((HALF, 128), jnp.float32),
    )(lutT, lutT)


def _body(lutp_hbm, xT_hbm, outT_hbm, idxr_v, row_v, cb_v, g_v, ob_v, sem):
    wid = lax.axis_index("s") * NC + lax.axis_index("c")
    iota16 = jax.lax.iota(jnp.int32, 16)

    @pl.loop(0, PER_W)
    def _(cc):
        c = wid * PER_W + cc
        h = c % HIST
        b0 = (c // HIST) * BBLK
        pltpu.sync_copy(xT_hbm.at[h, pl.ds(b0, BBLK)], idxr_v)
        for k in range(BBLK // 16):
            v = idxr_v[pl.ds(k * 16, 16)]
            row_v[pl.ds(k * 16, 16)] = jnp.where(v >= HALF, v - HALF, v)
            cb_v[pl.ds(k * 16, 16)] = jnp.where(
                v >= HALF, jnp.int32(D), jnp.int32(0))
        pltpu.async_copy(lutp_hbm.at[row_v], g_v, sem).wait()
        cbs = [cb_v[pl.ds(jj * 16, 16)] for jj in range(BBLK // 16)]
        rws = [jj * 16 + iota16 for jj in range(BBLK // 16)]

        @pl.loop(0, D)
        def _(d):
            for jj in range(BBLK // 16):
                ob_v[d, pl.ds(jj * 16, 16)] = plsc.load_gather(
                    g_v, [rws[jj], cbs[jj] + d])

        pltpu.sync_copy(ob_v, outT_hbm.at[h, :, pl.ds(b0, BBLK)])


@jax.jit
def _call(lutp, xT):
    mesh = plsc.VectorSubcoreMesh(core_axis_name="c", subcore_axis_name="s",
                                  num_cores=NC, num_subcores=NS)
    return pl.kernel(
        _body,
        out_type=jax.ShapeDtypeStruct((HIST, D, BATCH), jnp.float32),
        mesh=mesh,
        scratch_types=[
            pltpu.VMEM((BBLK,), jnp.int32),
            pltpu.VMEM((BBLK,), jnp.int32),
            pltpu.VMEM((BBLK,), jnp.int32),
            pltpu.VMEM((BBLK, 128), jnp.float32),
            pltpu.VMEM((D, BBLK), jnp.float32),
            pltpu.SemaphoreType.DMA,
        ],
        compiler_params=pltpu.CompilerParams(use_tc_tiling_on_sc=True,
                                             needs_layout_passes=False),
    )(lutp, xT)


def kernel(x, lut):
    lutp = _prep(lut.T)
    outT = _call(lutp, x.T)
    return outT.transpose(2, 0, 1)


# speedup vs baseline: 1.8748x; 1.8748x over previous
import jax
import jax.numpy as jnp
from jax import lax
from jax.experimental import pallas as pl
from jax.experimental.pallas import tpu as pltpu
from jax.experimental.pallas import tpu_sc as plsc

BATCH = 16384
HIST = 50
EMBED_DIM = 64
TOTAL = BATCH * HIST
NUM_CORES = 2
NUM_SUBCORES = 16
NW = NUM_CORES * NUM_SUBCORES
PER_W = TOTAL // NW
CHUNK = 256
GROUPS = PER_W // CHUNK
K = 2
NSS = GROUPS // K


def _emb_body(lut_hbm, idx_hbm, out_hbm, idx_v, rows_a, rows_b,
              gsem_a, gsem_b, osem_a, osem_b):
    wid = lax.axis_index("s") * NUM_CORES + lax.axis_index("c")
    base = wid * PER_W
    pltpu.sync_copy(idx_hbm.at[wid], idx_v)

    rows = (rows_a, rows_b)
    gsem = (gsem_a, gsem_b)
    osem = (osem_a, osem_b)

    def fire_gather(ss, p):
        for b in range(K):
            pltpu.async_copy(lut_hbm.at[idx_v.at[ss * K + b]], rows[p].at[b],
                             gsem[p])

    def drain_gather(p):
        for b in range(K):
            pltpu.make_async_copy(lut_hbm.at[idx_v.at[0]], rows[p].at[b],
                                  gsem[p]).wait()

    def fire_write(ss, p):
        for b in range(K):
            j = ss * K + b
            pltpu.async_copy(rows[p].at[b],
                             out_hbm.at[pl.ds(base + j * CHUNK, CHUNK)],
                             osem[p])

    def drain_write(p):
        for b in range(K):
            pltpu.make_async_copy(rows[p].at[b],
                                  out_hbm.at[pl.ds(base, CHUNK)],
                                  osem[p]).wait()

    fire_gather(0, 0)
    drain_gather(0)
    fire_write(0, 0)
    fire_gather(1, 1)

    @pl.loop(0, NSS - 2, step=2)
    def _(i):
        for q in (0, 1):
            s = i + 1 + q
            p = (1 + q) % 2
            drain_gather(p)
            fire_write(s, p)
            drain_write(1 - p)
            fire_gather(s + 1, 1 - p)

    drain_gather(1)
    fire_write(NSS - 1, 1)
    drain_write(0)
    drain_write(1)


@jax.jit
def _emb_call(lut, idx):
    mesh = plsc.VectorSubcoreMesh(
        core_axis_name="c", subcore_axis_name="s",
        num_cores=NUM_CORES, num_subcores=NUM_SUBCORES,
    )
    return pl.kernel(
        _emb_body,
        out_type=jax.ShapeDtypeStruct((TOTAL, EMBED_DIM), jnp.float32),
        mesh=mesh,
        scratch_types=[
            pltpu.VMEM((GROUPS, CHUNK), jnp.int32),
            pltpu.VMEM((K, CHUNK, EMBED_DIM), jnp.float32),
            pltpu.VMEM((K, CHUNK, EMBED_DIM), jnp.float32),
            pltpu.SemaphoreType.DMA,
            pltpu.SemaphoreType.DMA,
            pltpu.SemaphoreType.DMA,
            pltpu.SemaphoreType.DMA,
        ],
        compiler_params=pltpu.CompilerParams(use_tc_tiling_on_sc=False),
    )(lut, idx)


def kernel(x, lut):
    idx = x.reshape(NW, GROUPS, CHUNK)
    out = _emb_call(lut, idx)
    return out.reshape(BATCH, HIST, EMBED_DIM)
